# Initial kernel scaffold; baseline (speedup 1.0000x reference)
import jax
import jax.numpy as jnp
from jax.experimental import pallas as pl


def kernel(x, edge_index, batch, params, final_params):
    raise NotImplementedError("write your pallas kernel here")


# speedup vs baseline: 5.8027x; 5.8027x over previous
import functools

import jax
import jax.numpy as jnp
from jax import lax
from jax.experimental import pallas as pl
from jax.experimental.pallas import tpu as pltpu
from jax.experimental.pallas import tpu_sc as plsc

_NC = 2
_NS = 16
_NW = _NC * _NS
_LANES = 16
_K = 128


def _chunk_sizes(total, step):
    out = []
    while total > 0:
        out.append(min(step, total))
        total -= out[-1]
    return out


@functools.lru_cache(maxsize=None)
def _make_sc_agg(N, D, CH):
    NPAD = -(-(N + 1) // (_NS * 8)) * (_NS * 8)
    RZ = NPAD // _NS
    mesh = plsc.VectorSubcoreMesh(core_axis_name="c", subcore_axis_name="s")

    @functools.partial(
        pl.kernel,
        mesh=mesh,
        out_type=jax.ShapeDtypeStruct((_NC, NPAD, D), jnp.float32),
        scratch_types=[
            pltpu.VMEM((CH, _K), jnp.int32),
            pltpu.VMEM((2, _K), jnp.int32),
            pltpu.VMEM((2, _K), jnp.int32),
            pltpu.VMEM((_K, D), jnp.float32),
            pltpu.VMEM((_K, D), jnp.float32),
            pltpu.VMEM_SHARED((NPAD, D), jnp.float32),
            pltpu.SemaphoreType.DMA,
            pltpu.SemaphoreType.DMA,
        ],
    )
    def sc_agg(x_hbm, pk_hbm, out_hbm, pk_v, src2, dst2, buf0, buf1,
               acc, sem0, sem1):
        c = lax.axis_index("c")
        s = lax.axis_index("s")
        wid = c * _NS + s

        pltpu.sync_copy(pk_hbm.at[wid], pk_v)
        nkv = _K // _LANES

        def _unpack(j, par):
            for q in range(nkv):
                v = pk_v[j, pl.ds(q * _LANES, _LANES)]
                src2[par, pl.ds(q * _LANES, _LANES)] = (
                    lax.shift_right_logical(v, 14))
                dst2[par, pl.ds(q * _LANES, _LANES)] = (
                    lax.bitwise_and(v, 16383))

        zeros = jnp.zeros((_LANES,), jnp.float32)
        nvec = D // _LANES

        def _zb(i, _):
            buf0[i // nvec, pl.ds((i % nvec) * _LANES, _LANES)] = zeros
            return 0

        lax.fori_loop(0, _K * nvec, _zb, 0)
        off = 0
        for sz in _chunk_sizes(RZ, _K):
            pltpu.sync_copy(buf0.at[pl.ds(0, sz)],
                            acc.at[pl.ds(s * RZ + off, sz)])
            off += sz
        plsc.subcore_barrier()

        def _gather(par, buf, sem):
            pltpu.async_copy(x_hbm.at[src2.at[par]], buf, sem)

        def _wait(buf, sem):
            pltpu.make_async_copy(x_hbm.at[pl.ds(0, _K)], buf, sem).wait()

        _unpack(0, 0)
        _gather(0, buf0, sem0)

        def _step(t, _):
            j0 = 2 * t

            @pl.when(j0 + 1 < CH)
            def _():
                _unpack(j0 + 1, 1)
                _gather(1, buf1, sem1)

            _wait(buf0, sem0)
            pltpu.sync_copy(buf0, acc.at[dst2.at[0]], add=True)

            @pl.when(j0 + 2 < CH)
            def _():
                _unpack(j0 + 2, 0)
                _gather(0, buf0, sem0)

            @pl.when(j0 + 1 < CH)
            def _():
                _wait(buf1, sem1)
                pltpu.sync_copy(buf1, acc.at[dst2.at[1]], add=True)

            return 0

        lax.fori_loop(0, (CH + 1) // 2, _step, 0)
        plsc.subcore_barrier()

        off = 0
        for sz in _chunk_sizes(RZ, _K):
            pltpu.sync_copy(acc.at[pl.ds(s * RZ + off, sz)],
                            out_hbm.at[c, pl.ds(s * RZ + off, sz)])
            off += sz

    return sc_agg


def _mlp_body(parts_ref, x_ref, w1_ref, b1_ref, g_ref, be_ref, w2_ref,
              b2_ref, o_ref):
    n = x_ref.shape[0]
    p = parts_ref[...]
    h0 = p[0, :n] + p[1, :n] + x_ref[...]
    h1 = jnp.dot(h0, w1_ref[...], preferred_element_type=jnp.float32)
    h1 = h1 + b1_ref[...]
    mu = jnp.mean(h1, axis=0, keepdims=True)
    var = jnp.mean(jnp.square(h1 - mu), axis=0, keepdims=True)
    h1 = (h1 - mu) * lax.rsqrt(var + 1e-5) * g_ref[...] + be_ref[...]
    h1 = jnp.maximum(h1, 0.0)
    h2 = jnp.dot(h1, w2_ref[...], preferred_element_type=jnp.float32)
    h2 = h2 + b2_ref[...]
    o_ref[...] = jnp.maximum(h2, 0.0)


def _pool_body(x_ref, batch_ref, w3_ref, b3_ref, w4_ref, b4_ref, o_ref, *,
               G):
    n = x_ref.shape[0]
    gid = lax.broadcasted_iota(jnp.int32, (G, n), 0)
    onehot = jnp.where(gid == batch_ref[...], 1.0, 0.0)
    seg = jnp.dot(onehot, x_ref[...], preferred_element_type=jnp.float32)
    cnt = jnp.sum(onehot, axis=1, keepdims=True)
    pooled = seg / jnp.maximum(cnt, 1.0)
    h = jnp.dot(pooled, w3_ref[...], preferred_element_type=jnp.float32)
    h = jnp.maximum(h + b3_ref[...], 0.0)
    o_ref[...] = jnp.dot(h, w4_ref[...],
                         preferred_element_type=jnp.float32) + b4_ref[...]


def kernel(x, edge_index, batch, params, final_params):
    N, D = x.shape
    G = 64
    src = edge_index[0].astype(jnp.int32)
    dst = edge_index[1].astype(jnp.int32)
    E = src.shape[0]

    EP = _NW * _K * (-(-E // (_NW * _K)))
    CH = EP // (_NW * _K)
    pk = src * 16384 + dst
    pk_p = jnp.concatenate(
        [pk, jnp.full((EP - E,), N, jnp.int32)]).reshape(_NW, CH, _K)

    sc_agg = _make_sc_agg(N, D, CH)

    H = params[0][0].shape[1]
    mlp_call = pl.pallas_call(
        _mlp_body,
        out_shape=jax.ShapeDtypeStruct((N, H), jnp.float32),
    )

    h = x
    for (W1, b1, gamma, beta, W2, b2) in params:
        parts = sc_agg(h, pk_p)
        h = mlp_call(parts, h, W1, b1.reshape(1, -1), gamma.reshape(1, -1),
                     beta.reshape(1, -1), W2, b2.reshape(1, -1))

    W3, b3, W4, b4 = final_params
    OUT = W4.shape[1]
    pool_call = pl.pallas_call(
        functools.partial(_pool_body, G=G),
        out_shape=jax.ShapeDtypeStruct((G, OUT), jnp.float32),
    )
    return pool_call(h, batch.reshape(1, -1).astype(jnp.int32), W3,
                     b3.reshape(1, -1), W4, b4.reshape(1, -1))


# speedup vs baseline: 9.5060x; 1.6382x over previous
import functools

import jax
import jax.numpy as jnp
from jax import lax
from jax.experimental import pallas as pl
from jax.experimental.pallas import tpu as pltpu
from jax.experimental.pallas import tpu_sc as plsc

_NC = 2
_NS = 16
_NW = _NC * _NS
_LANES = 16
_K = 128


def _chunk_sizes(total, step):
    out = []
    while total > 0:
        out.append(min(step, total))
        total -= out[-1]
    return out


@functools.lru_cache(maxsize=None)
def _make_sc_agg(N, D, CH0, CH1):
    CH = max(CH0, CH1)
    NPAD = -(-(N + 1) // (_NS * 8)) * (_NS * 8)
    RZ = NPAD // _NS
    mesh = plsc.VectorSubcoreMesh(core_axis_name="c", subcore_axis_name="s")

    @functools.partial(
        pl.kernel,
        mesh=mesh,
        out_type=jax.ShapeDtypeStruct((_NC, NPAD, D), jnp.float32),
        scratch_types=[
            pltpu.VMEM((CH, _K), jnp.int32),
            pltpu.VMEM((2, _K), jnp.int32),
            pltpu.VMEM((2, _K), jnp.int32),
            pltpu.VMEM((_K, D), jnp.float32),
            pltpu.VMEM((_K, D), jnp.float32),
            pltpu.VMEM_SHARED((NPAD, D), jnp.float32),
            pltpu.SemaphoreType.DMA,
            pltpu.SemaphoreType.DMA,
        ],
    )
    def sc_agg(x_hbm, pk_hbm, out_hbm, pk_v, src2, dst2, buf0, buf1,
               acc, sem0, sem1):
        c = lax.axis_index("c")
        s = lax.axis_index("s")
        wid = c * _NS + s

        pltpu.sync_copy(pk_hbm.at[wid], pk_v)
        nkv = _K // _LANES

        def _unpack(j, par):
            for q in range(nkv):
                v = pk_v[j, pl.ds(q * _LANES, _LANES)]
                src2[par, pl.ds(q * _LANES, _LANES)] = (
                    lax.shift_right_logical(v, 14))
                dst2[par, pl.ds(q * _LANES, _LANES)] = (
                    lax.bitwise_and(v, 16383))

        zeros = jnp.zeros((_LANES,), jnp.float32)
        nvec = D // _LANES

        def _zb(i, _):
            buf0[i // nvec, pl.ds((i % nvec) * _LANES, _LANES)] = zeros
            return 0

        lax.fori_loop(0, _K * nvec, _zb, 0)
        off = 0
        for sz in _chunk_sizes(RZ, _K):
            pltpu.sync_copy(buf0.at[pl.ds(0, sz)],
                            acc.at[pl.ds(s * RZ + off, sz)])
            off += sz
        plsc.subcore_barrier()

        def _gather(par, buf, sem):
            pltpu.async_copy(x_hbm.at[src2.at[par]], buf, sem)

        def _wait(buf, sem):
            pltpu.make_async_copy(x_hbm.at[pl.ds(0, _K)], buf, sem).wait()

        CHc = jnp.where(c == 0, CH0, CH1)
        _unpack(0, 0)
        _gather(0, buf0, sem0)

        def _step(t, _):
            j0 = 2 * t

            @pl.when(j0 + 1 < CHc)
            def _():
                _unpack(j0 + 1, 1)
                _gather(1, buf1, sem1)

            _wait(buf0, sem0)
            pltpu.sync_copy(buf0, acc.at[dst2.at[0]], add=True)

            @pl.when(j0 + 2 < CHc)
            def _():
                _unpack(j0 + 2, 0)
                _gather(0, buf0, sem0)

            @pl.when(j0 + 1 < CHc)
            def _():
                _wait(buf1, sem1)
                pltpu.sync_copy(buf1, acc.at[dst2.at[1]], add=True)

            return 0

        lax.fori_loop(0, (CHc + 1) // 2, _step, 0)
        plsc.subcore_barrier()

        off = 0
        for sz in _chunk_sizes(RZ, _K):
            pltpu.sync_copy(acc.at[pl.ds(s * RZ + off, sz)],
                            out_hbm.at[c, pl.ds(s * RZ + off, sz)])
            off += sz

    return sc_agg


def _mlp_body(parts_ref, x_ref, w1_ref, b1_ref, g_ref, be_ref, w2_ref,
              b2_ref, o_ref):
    n = x_ref.shape[0]
    p = parts_ref[...]
    h0 = p[0, :n] + p[1, :n] + x_ref[...]
    h1 = jnp.dot(h0, w1_ref[...], preferred_element_type=jnp.float32)
    h1 = h1 + b1_ref[...]
    mu = jnp.mean(h1, axis=0, keepdims=True)
    var = jnp.mean(jnp.square(h1 - mu), axis=0, keepdims=True)
    h1 = (h1 - mu) * lax.rsqrt(var + 1e-5) * g_ref[...] + be_ref[...]
    h1 = jnp.maximum(h1, 0.0)
    h2 = jnp.dot(h1, w2_ref[...], preferred_element_type=jnp.float32)
    h2 = h2 + b2_ref[...]
    o_ref[...] = jnp.maximum(h2, 0.0)


def _pool_body(x_ref, batch_ref, w3_ref, b3_ref, w4_ref, b4_ref, o_ref, *,
               G):
    n = x_ref.shape[0]
    gid = lax.broadcasted_iota(jnp.int32, (G, n), 0)
    onehot = jnp.where(gid == batch_ref[...], 1.0, 0.0)
    seg = jnp.dot(onehot, x_ref[...], preferred_element_type=jnp.float32)
    cnt = jnp.sum(onehot, axis=1, keepdims=True)
    pooled = seg / jnp.maximum(cnt, 1.0)
    h = jnp.dot(pooled, w3_ref[...], preferred_element_type=jnp.float32)
    h = jnp.maximum(h + b3_ref[...], 0.0)
    o_ref[...] = jnp.dot(h, w4_ref[...],
                         preferred_element_type=jnp.float32) + b4_ref[...]


def kernel(x, edge_index, batch, params, final_params):
    N, D = x.shape
    G = 64
    src = edge_index[0].astype(jnp.int32)
    dst = edge_index[1].astype(jnp.int32)
    E = src.shape[0]

    TCH = -(-E // _K)
    per_w = -(-TCH // _NS)
    CH0 = max(1, min(per_w - 1, round(per_w * 0.715)))
    CH1 = per_w - CH0
    EP = _NS * per_w * _K
    pk = src * 16384 + dst
    pkc = jnp.concatenate(
        [pk, jnp.full((EP - E,), N, jnp.int32)]).reshape(_NS * per_w, _K)
    blocks = [pkc[w * CH0:(w + 1) * CH0] for w in range(_NS)]
    base1 = _NS * CH0
    fill = jnp.zeros((CH0 - CH1, _K), jnp.int32)
    for w in range(_NS):
        blocks.append(jnp.concatenate(
            [pkc[base1 + w * CH1:base1 + (w + 1) * CH1], fill]))
    pk_p = jnp.stack(blocks)

    sc_agg = _make_sc_agg(N, D, CH0, CH1)

    H = params[0][0].shape[1]
    mlp_call = pl.pallas_call(
        _mlp_body,
        out_shape=jax.ShapeDtypeStruct((N, H), jnp.float32),
    )

    h = x
    for (W1, b1, gamma, beta, W2, b2) in params:
        parts = sc_agg(h, pk_p)
        h = mlp_call(parts, h, W1, b1.reshape(1, -1), gamma.reshape(1, -1),
                     beta.reshape(1, -1), W2, b2.reshape(1, -1))

    W3, b3, W4, b4 = final_params
    OUT = W4.shape[1]
    pool_call = pl.pallas_call(
        functools.partial(_pool_body, G=G),
        out_shape=jax.ShapeDtypeStruct((G, OUT), jnp.float32),
    )
    return pool_call(h, batch.reshape(1, -1).astype(jnp.int32), W3,
                     b3.reshape(1, -1), W4, b4.reshape(1, -1))
